# baseline (speedup 1.0000x reference)
import functools

import jax
import jax.numpy as jnp
from jax import lax
from jax.experimental import pallas as pl
from jax.experimental.pallas import tpu as pltpu
from jax.experimental.pallas import tpu_sc as plsc

N = 10000
E = 160000
D_IN = 256
H = 8
DH = 32

NC = 2
NS = 16
TPH = 4
HPC = H // NC
EPT = E // TPH
CH = 80
NCHUNK = EPT // CH
SUB = 80
NSUB = CH // SUB
GPS = SUB // 16
SC_G = 10
NSUPER = NCHUNK // SC_G
NB = N // 16
ZROWS = 20
OROWS = 500
ACC_ROWS_PER_TILE = HPC * N // NS


def _tc_prep_body(x_ref, wt_ref, sa_ref, da_ref, hh_ref, sc_ref):
    hb = jnp.dot(x_ref[...], wt_ref[...], preferred_element_type=jnp.float32)
    cols = []
    for h in range(H):
        hslice = hb[:, h * DH:(h + 1) * DH]
        hh_ref[h] = hslice.astype(jnp.bfloat16)
        cols.append((hslice * sa_ref[h][None, :]).sum(axis=-1))
    for h in range(H):
        hslice = hb[:, h * DH:(h + 1) * DH]
        cols.append((hslice * da_ref[h][None, :]).sum(axis=-1))
    sc_ref[...] = jnp.stack(cols, axis=-1)


def _tc_prep(x, wt, sa, da):
    blk = 2000
    return pl.pallas_call(
        _tc_prep_body,
        grid=(N // blk,),
        in_specs=[
            pl.BlockSpec((blk, D_IN), lambda i: (i, 0)),
            pl.BlockSpec((D_IN, D_IN), lambda i: (0, 0)),
            pl.BlockSpec((H, DH), lambda i: (0, 0)),
            pl.BlockSpec((H, DH), lambda i: (0, 0)),
        ],
        out_specs=[
            pl.BlockSpec((H, blk, DH), lambda i: (0, i, 0)),
            pl.BlockSpec((blk, 2 * H), lambda i: (i, 0)),
        ],
        out_shape=[
            jax.ShapeDtypeStruct((H, N, DH), jnp.bfloat16),
            jax.ShapeDtypeStruct((N, 2 * H), jnp.float32),
        ],
    )(x, wt, sa, da)


def _sc_body(h_flat, scores, colg, rows_s, num_out, den_out,
             src_t, dst_t, den_t, idxg, idxs, e_v, rows, scl, zbuf, acc,
             semg0, semg1, sems0, sems1):
    c = lax.axis_index("c")
    s = lax.axis_index("s")
    hl = s // TPH
    q = lax.rem(s, TPH)
    head = c * HPC + hl
    zero16 = jnp.zeros((16,), jnp.float32)
    headn = jnp.full((16,), head * N, jnp.int32)
    hln = jnp.full((16,), hl * N, jnp.int32)

    pltpu.sync_copy(scores.at[0, head], src_t)
    pltpu.sync_copy(scores.at[1, head], dst_t)

    gdn = lax.GatherDimensionNumbers(offset_dims=(), collapsed_slice_dims=(0,),
                                     start_index_map=(0,))
    def _perm(v, perm):
        return lax.gather(v, perm[:, None], gdn, slice_sizes=(1,),
                          mode=lax.GatherScatterMode.PROMISE_IN_BOUNDS)
    def _allmax(v):
        for k in (8, 4, 2, 1):
            perm = lax.iota(jnp.int32, 16) ^ k
            v = jnp.maximum(v, _perm(v, perm))
        return v
    def _mx(i, m):
        return jnp.maximum(m, src_t[pl.ds(i * 16, 16)])
    smax = _allmax(lax.fori_loop(0, NB, _mx, jnp.full((16,), -jnp.inf, jnp.float32)))
    def _mx2(i, m):
        return jnp.maximum(m, dst_t[pl.ds(i * 16, 16)])
    dmax = _allmax(lax.fori_loop(0, NB, _mx2, jnp.full((16,), -jnp.inf, jnp.float32)))
    shift = jnp.maximum(smax + dmax, 0.0)

    def _zd(i, _):
        den_t[pl.ds(i * 16, 16)] = zero16
        return 0
    lax.fori_loop(0, NB, _zd, 0)

    def _zb(i, _):
        zbuf[i, pl.ds(0, 16)] = zero16
        zbuf[i, pl.ds(16, 16)] = zero16
        return 0
    lax.fori_loop(0, ZROWS, _zb, 0)
    def _zacc(i, _):
        pltpu.async_copy(
            zbuf, acc.at[pl.ds(s * ACC_ROWS_PER_TILE + i * ZROWS, ZROWS)], semg0)
        return 0
    lax.fori_loop(0, ACC_ROWS_PER_TILE // ZROWS, _zacc, 0)
    def _zacc_w(i, _):
        pltpu.make_async_copy(
            zbuf, acc.at[pl.ds(s * ACC_ROWS_PER_TILE + i * ZROWS, ZROWS)],
            semg0).wait()
        return 0
    lax.fori_loop(0, ACC_ROWS_PER_TILE // ZROWS, _zacc_w, 0)
    plsc.subcore_barrier()

    semg = (semg0, semg1)
    sems = (sems0, sems1)

    def _load_idx(pb, sbl):
        pltpu.sync_copy(colg.at[head, sbl], idxg.at[pb])
        pltpu.sync_copy(rows_s.at[head, sbl], idxs.at[pb])

    def _fire_g(pd, pb, lci):
        for j in range(NSUB):
            pltpu.async_copy(h_flat.at[idxg.at[pb, lci * NSUB + j]],
                             rows.at[pd, j], semg[pd])

    def _drain_g(pd, pb, lci):
        for j in range(NSUB):
            pltpu.make_async_copy(h_flat.at[idxg.at[pb, lci * NSUB + j]],
                                  rows.at[pd, j], semg[pd]).wait()

    def _fire_s(pd, pb, lci):
        for j in range(NSUB):
            pltpu.async_copy(scl.at[pd, j],
                             acc.at[idxs.at[pb, lci * NSUB + j]], sems[pd],
                             add=True)

    def _drain_s(pd, pb, lci):
        for j in range(NSUB):
            pltpu.make_async_copy(scl.at[pd, j],
                                  acc.at[idxs.at[pb, lci * NSUB + j]],
                                  sems[pd]).wait()

    def _score(pb, lci):
        def _score_a(a, _a):
            def _score_t(t, _t):
                rs = idxs[pb, lci * NSUB + a, pl.ds(t * 16, 16)]
                cg = idxg[pb, lci * NSUB + a, pl.ds(t * 16, 16)]
                r = rs - hln
                col16 = cg - headn
                sv = plsc.load_gather(src_t, [r])
                dv = plsc.load_gather(dst_t, [col16])
                t_ = sv + dv
                sc = jnp.maximum(t_, 0.2 * t_) - shift
                e = jnp.exp(sc)
                e_v[pl.ds(a * SUB + t * 16, 16)] = e
                plsc.addupdate_scatter(den_t, [r], e)
                return 0
            lax.fori_loop(0, GPS, _score_t, 0)
            return 0
        lax.fori_loop(0, NSUB, _score_a, 0)

    def _scale(pd):
        for a in range(NSUB):
            def _scale_b(bb, _b, _a=a):
                eb = plsc.load_gather(
                    e_v, [jnp.full((16,), _a * SUB + bb, jnp.int32)])
                rv = rows[pd, _a, bb]
                ev, od = plsc.unpack(rv, format=plsc.PackFormat.INTERLEAVED,
                                     preferred_element_type=jnp.float32)
                scl[pd, _a, bb, pl.ds(0, 16)] = ev * eb
                scl[pd, _a, bb, pl.ds(16, 16)] = od * eb
                return 0
            lax.fori_loop(0, SUB, _scale_b, 0, unroll=4)

    def _step(pb, lci, sbl_next, first=False):
        pd = lci % 2
        po = 1 - pd
        _score(pb, lci)
        _drain_g(pd, pb, lci)
        _scale(pd)
        _fire_s(pd, pb, lci)
        if not first:
            prev_pb, prev_lci = (pb, lci - 1) if lci >= 1 else (1 - pb, SC_G - 1)
            _drain_s(po, prev_pb, prev_lci)
        if lci == SC_G - 2:
            _load_idx(1 - pb, sbl_next)
        next_pb, next_lci = (pb, lci + 1) if lci < SC_G - 1 else (1 - pb, 0)
        _fire_g(po, next_pb, next_lci)

    sbase = q * NSUPER
    _load_idx(0, sbase)
    _fire_g(0, 0, 0)
    for lci in range(SC_G):
        _step(0, lci, sbase + 1, first=(lci == 0))

    def _super(sb_off, _):
        sb = sb_off + 1
        pb = lax.rem(sb, 2)
        nxt = sbase + jnp.minimum(sb + 1, NSUPER - 1)
        for lci in range(SC_G):
            _step(pb, lci, nxt)
        return 0
    lax.fori_loop(0, NSUPER - 1, _super, 0)

    _drain_s(1, 0, SC_G - 1)
    _drain_g(0, 1, 0)

    plsc.subcore_barrier()

    out0 = hl * N + q * (N // TPH)
    nblk = N // TPH // OROWS
    def _out(i, _):
        pltpu.async_copy(acc.at[pl.ds(out0 + i * OROWS, OROWS)],
                         num_out.at[head, q * nblk + i], semg1)
        return 0
    lax.fori_loop(0, nblk, _out, 0)
    pltpu.sync_copy(den_t, den_out.at[head, q, 0])
    def _out_w(i, _):
        pltpu.make_async_copy(acc.at[pl.ds(out0 + i * OROWS, OROWS)],
                              num_out.at[head, q * nblk + i], semg1).wait()
        return 0
    lax.fori_loop(0, nblk, _out_w, 0)


def _sc_edge(h_flat, scores, colg, rows_s):
    mesh = plsc.VectorSubcoreMesh(core_axis_name="c", subcore_axis_name="s",
                                  num_cores=NC, num_subcores=NS)
    f = functools.partial(
        pl.kernel,
        compiler_params=pltpu.CompilerParams(use_tc_tiling_on_sc=False,
                                             needs_layout_passes=False),
        out_type=[
            jax.ShapeDtypeStruct((H, N // OROWS, OROWS, DH), jnp.float32),
            jax.ShapeDtypeStruct((H, TPH, 1, N), jnp.float32),
        ],
        mesh=mesh,
        scratch_types=[
            pltpu.VMEM((N,), jnp.float32),
            pltpu.VMEM((N,), jnp.float32),
            pltpu.VMEM((N,), jnp.float32),
            pltpu.VMEM((2, SC_G * NSUB, SUB), jnp.int32),
            pltpu.VMEM((2, SC_G * NSUB, SUB), jnp.int32),
            pltpu.VMEM((CH,), jnp.float32),
            pltpu.VMEM((2, NSUB, SUB, DH), jnp.bfloat16),
            pltpu.VMEM((2, NSUB, SUB, DH), jnp.float32),
            pltpu.VMEM((ZROWS, DH), jnp.float32),
            pltpu.VMEM_SHARED((HPC * N, DH), jnp.float32),
            pltpu.SemaphoreType.DMA,
            pltpu.SemaphoreType.DMA,
            pltpu.SemaphoreType.DMA,
            pltpu.SemaphoreType.DMA,
        ],
    )(_sc_body)
    return f(h_flat, scores, colg, rows_s)


def _tc_final_body(num_ref, den_ref, out_ref):
    blk = out_ref.shape[0]
    d = den_ref[...].reshape(blk, H, TPH).sum(axis=-1)
    dsafe = jnp.where(d == 0.0, 1.0, d)
    def _unperm(x):
        return x.reshape(blk, 2, DH // 2).transpose(0, 2, 1).reshape(blk, DH)
    parts = [_unperm(num_ref[h]) / dsafe[:, h][:, None] for h in range(H)]
    out_ref[...] = jnp.concatenate(parts, axis=-1)


def _tc_final(num, den):
    blk = 1000
    return pl.pallas_call(
        _tc_final_body,
        grid=(N // blk,),
        in_specs=[
            pl.BlockSpec((H, blk, DH), lambda i: (0, i, 0)),
            pl.BlockSpec((blk, H * TPH), lambda i: (i, 0)),
        ],
        out_specs=pl.BlockSpec((blk, H * DH), lambda i: (i, 0)),
        out_shape=jax.ShapeDtypeStruct((N, H * DH), jnp.float32),
    )(num, den)


def kernel(x, edge_indices, W, src_attn, dst_attn):
    row = edge_indices[0]
    col = edge_indices[1]
    wt = W.T
    sa = src_attn.reshape(H, DH)
    da = dst_attn.reshape(H, DH)
    h_heads, scores_nk = _tc_prep(x, wt, sa, da)
    scores = scores_nk.T.reshape(2, H, N)
    h_flat = h_heads.reshape(H * N, DH)
    heads = jnp.arange(H, dtype=jnp.int32)
    nblks = E // SUB // (SC_G * NSUB)
    colg = (col[None, :] + heads[:, None] * N).reshape(H, nblks, SC_G * NSUB, SUB)
    rows_s = (row[None, :] + (heads[:, None] % TPH) * N).reshape(H, nblks, SC_G * NSUB, SUB)
    num, den = _sc_edge(h_flat, scores, colg, rows_s)
    den_nk = den.reshape(H * TPH, N).T
    return _tc_final(num.reshape(H, N, DH), den_nk)


# speedup vs baseline: 1.8540x; 1.8540x over previous
import functools

import jax
import jax.numpy as jnp
from jax import lax
from jax.experimental import pallas as pl
from jax.experimental.pallas import tpu as pltpu
from jax.experimental.pallas import tpu_sc as plsc

N = 10000
E = 160000
D_IN = 256
H = 8
DH = 32

NC = 2
NS = 16
TPH = 4
HPC = H // NC
EPT = E // TPH
CH = 160
NCHUNK = EPT // CH
SUB = 80
NSUB = CH // SUB
GPS = SUB // 16
SC_G = 10
NSUPER = NCHUNK // SC_G
NB = N // 16
ZROWS = 50
OROWS = 500
ACC_ROWS_PER_TILE = HPC * N // NS


def _tc_prep_body(x_ref, wt_ref, sa_ref, da_ref, hh_ref, sc_ref):
    hb = jnp.dot(x_ref[...], wt_ref[...], preferred_element_type=jnp.float32)
    cols = []
    for h in range(H):
        hslice = hb[:, h * DH:(h + 1) * DH]
        hh_ref[h] = hslice
        cols.append((hslice * sa_ref[h][None, :]).sum(axis=-1))
    for h in range(H):
        hslice = hb[:, h * DH:(h + 1) * DH]
        cols.append((hslice * da_ref[h][None, :]).sum(axis=-1))
    sc_ref[...] = jnp.stack(cols, axis=-1)


def _tc_prep(x, wt, sa, da):
    blk = 1000
    return pl.pallas_call(
        _tc_prep_body,
        grid=(N // blk,),
        in_specs=[
            pl.BlockSpec((blk, D_IN), lambda i: (i, 0)),
            pl.BlockSpec((D_IN, D_IN), lambda i: (0, 0)),
            pl.BlockSpec((H, DH), lambda i: (0, 0)),
            pl.BlockSpec((H, DH), lambda i: (0, 0)),
        ],
        out_specs=[
            pl.BlockSpec((H, blk, DH), lambda i: (0, i, 0)),
            pl.BlockSpec((blk, 2 * H), lambda i: (i, 0)),
        ],
        out_shape=[
            jax.ShapeDtypeStruct((H, N, DH), jnp.float32),
            jax.ShapeDtypeStruct((N, 2 * H), jnp.float32),
        ],
    )(x, wt, sa, da)


def _sc_body(h_flat, scores, colg, rows_s, num_out, den_out,
             src_t, dst_t, den_t, idxg, idxs, e_v, rows, zbuf, acc,
             semg0, semg1, sems0, sems1):
    c = lax.axis_index("c")
    s = lax.axis_index("s")
    hl = s // TPH
    q = lax.rem(s, TPH)
    head = c * HPC + hl
    zero16 = jnp.zeros((16,), jnp.float32)
    headn = jnp.full((16,), head * N, jnp.int32)
    hln = jnp.full((16,), hl * N, jnp.int32)

    pltpu.sync_copy(scores.at[0, head], src_t)
    pltpu.sync_copy(scores.at[1, head], dst_t)

    gdn = lax.GatherDimensionNumbers(offset_dims=(), collapsed_slice_dims=(0,),
                                     start_index_map=(0,))
    def _perm(v, perm):
        return lax.gather(v, perm[:, None], gdn, slice_sizes=(1,),
                          mode=lax.GatherScatterMode.PROMISE_IN_BOUNDS)
    def _allmax(v):
        for k in (8, 4, 2, 1):
            perm = lax.iota(jnp.int32, 16) ^ k
            v = jnp.maximum(v, _perm(v, perm))
        return v
    def _mx(i, m):
        return jnp.maximum(m, src_t[pl.ds(i * 16, 16)])
    smax = _allmax(lax.fori_loop(0, NB, _mx, jnp.full((16,), -jnp.inf, jnp.float32)))
    def _mx2(i, m):
        return jnp.maximum(m, dst_t[pl.ds(i * 16, 16)])
    dmax = _allmax(lax.fori_loop(0, NB, _mx2, jnp.full((16,), -jnp.inf, jnp.float32)))
    shift = jnp.maximum(smax + dmax, 0.0)

    def _zd(i, _):
        den_t[pl.ds(i * 16, 16)] = zero16
        return 0
    lax.fori_loop(0, NB, _zd, 0)

    def _zb(i, _):
        zbuf[i, pl.ds(0, 16)] = zero16
        zbuf[i, pl.ds(16, 16)] = zero16
        return 0
    lax.fori_loop(0, ZROWS, _zb, 0)
    def _zacc(i, _):
        pltpu.async_copy(
            zbuf, acc.at[pl.ds(s * ACC_ROWS_PER_TILE + i * ZROWS, ZROWS)], semg0)
        return 0
    lax.fori_loop(0, ACC_ROWS_PER_TILE // ZROWS, _zacc, 0)
    def _zacc_w(i, _):
        pltpu.make_async_copy(
            zbuf, acc.at[pl.ds(s * ACC_ROWS_PER_TILE + i * ZROWS, ZROWS)],
            semg0).wait()
        return 0
    lax.fori_loop(0, ACC_ROWS_PER_TILE // ZROWS, _zacc_w, 0)
    plsc.subcore_barrier()

    semg = (semg0, semg1)
    sems = (sems0, sems1)

    def _load_idx(pb, sbl):
        pltpu.sync_copy(colg.at[head, sbl], idxg.at[pb])
        pltpu.sync_copy(rows_s.at[head, sbl], idxs.at[pb])

    def _fire_g(pd, pb, lci):
        for j in range(NSUB):
            pltpu.async_copy(h_flat.at[idxg.at[pb, lci * NSUB + j]],
                             rows.at[pd, j], semg[pd])

    def _drain_g(pd, pb, lci):
        for j in range(NSUB):
            pltpu.make_async_copy(h_flat.at[idxg.at[pb, lci * NSUB + j]],
                                  rows.at[pd, j], semg[pd]).wait()

    def _fire_s(pd, pb, lci):
        for j in range(NSUB):
            pltpu.async_copy(rows.at[pd, j],
                             acc.at[idxs.at[pb, lci * NSUB + j]], sems[pd],
                             add=True)

    def _drain_s(pd, pb, lci):
        for j in range(NSUB):
            pltpu.make_async_copy(rows.at[pd, j],
                                  acc.at[idxs.at[pb, lci * NSUB + j]],
                                  sems[pd]).wait()

    def _score(pb, lci):
        def _score_a(a, _a):
            def _score_t(t, _t):
                rs = idxs[pb, lci * NSUB + a, pl.ds(t * 16, 16)]
                cg = idxg[pb, lci * NSUB + a, pl.ds(t * 16, 16)]
                r = rs - hln
                col16 = cg - headn
                sv = plsc.load_gather(src_t, [r])
                dv = plsc.load_gather(dst_t, [col16])
                t_ = sv + dv
                sc = jnp.maximum(t_, 0.2 * t_) - shift
                e = jnp.exp(sc)
                e_v[pl.ds(a * SUB + t * 16, 16)] = e
                plsc.addupdate_scatter(den_t, [r], e)
                return 0
            lax.fori_loop(0, GPS, _score_t, 0)
            return 0
        lax.fori_loop(0, NSUB, _score_a, 0)

    def _scale(pd):
        for a in range(NSUB):
            def _scale_b(bb, _b, _a=a):
                eb = plsc.load_gather(
                    e_v, [jnp.full((16,), _a * SUB + bb, jnp.int32)])
                rows[pd, _a, bb, pl.ds(0, 16)] = rows[pd, _a, bb, pl.ds(0, 16)] * eb
                rows[pd, _a, bb, pl.ds(16, 16)] = rows[pd, _a, bb, pl.ds(16, 16)] * eb
                return 0
            lax.fori_loop(0, SUB, _scale_b, 0, unroll=5)

    def _step(pb, lci, sbl_next, first=False):
        pd = lci % 2
        po = 1 - pd
        _score(pb, lci)
        _drain_g(pd, pb, lci)
        _scale(pd)
        _fire_s(pd, pb, lci)
        if not first:
            prev_pb, prev_lci = (pb, lci - 1) if lci >= 1 else (1 - pb, SC_G - 1)
            _drain_s(po, prev_pb, prev_lci)
        if lci == SC_G - 2:
            _load_idx(1 - pb, sbl_next)
        next_pb, next_lci = (pb, lci + 1) if lci < SC_G - 1 else (1 - pb, 0)
        _fire_g(po, next_pb, next_lci)

    sbase = q * NSUPER
    _load_idx(0, sbase)
    _fire_g(0, 0, 0)
    for lci in range(SC_G):
        _step(0, lci, sbase + 1, first=(lci == 0))

    def _super(sb_off, _):
        sb = sb_off + 1
        pb = lax.rem(sb, 2)
        nxt = sbase + jnp.minimum(sb + 1, NSUPER - 1)
        for lci in range(SC_G):
            _step(pb, lci, nxt)
        return 0
    lax.fori_loop(0, NSUPER - 1, _super, 0)

    _drain_s(1, 0, SC_G - 1)
    _drain_g(0, 1, 0)

    plsc.subcore_barrier()

    out0 = hl * N + q * (N // TPH)
    nblk = N // TPH // OROWS
    def _out(i, _):
        pltpu.async_copy(acc.at[pl.ds(out0 + i * OROWS, OROWS)],
                         num_out.at[head, q * nblk + i], semg1)
        return 0
    lax.fori_loop(0, nblk, _out, 0)
    pltpu.sync_copy(den_t, den_out.at[head, q, 0])
    def _out_w(i, _):
        pltpu.make_async_copy(acc.at[pl.ds(out0 + i * OROWS, OROWS)],
                              num_out.at[head, q * nblk + i], semg1).wait()
        return 0
    lax.fori_loop(0, nblk, _out_w, 0)


def _sc_edge(h_flat, scores, colg, rows_s):
    mesh = plsc.VectorSubcoreMesh(core_axis_name="c", subcore_axis_name="s",
                                  num_cores=NC, num_subcores=NS)
    f = functools.partial(
        pl.kernel,
        compiler_params=pltpu.CompilerParams(use_tc_tiling_on_sc=False,
                                             needs_layout_passes=False),
        out_type=[
            jax.ShapeDtypeStruct((H, N // OROWS, OROWS, DH), jnp.float32),
            jax.ShapeDtypeStruct((H, TPH, 1, N), jnp.float32),
        ],
        mesh=mesh,
        scratch_types=[
            pltpu.VMEM((N,), jnp.float32),
            pltpu.VMEM((N,), jnp.float32),
            pltpu.VMEM((N,), jnp.float32),
            pltpu.VMEM((2, SC_G * NSUB, SUB), jnp.int32),
            pltpu.VMEM((2, SC_G * NSUB, SUB), jnp.int32),
            pltpu.VMEM((CH,), jnp.float32),
            pltpu.VMEM((2, NSUB, SUB, DH), jnp.float32),
            pltpu.VMEM((ZROWS, DH), jnp.float32),
            pltpu.VMEM_SHARED((HPC * N, DH), jnp.float32),
            pltpu.SemaphoreType.DMA,
            pltpu.SemaphoreType.DMA,
            pltpu.SemaphoreType.DMA,
            pltpu.SemaphoreType.DMA,
        ],
    )(_sc_body)
    return f(h_flat, scores, colg, rows_s)


def _tc_final_body(num_ref, den_ref, out_ref):
    blk = out_ref.shape[0]
    d = den_ref[...].reshape(blk, H, TPH).sum(axis=-1)
    dsafe = jnp.where(d == 0.0, 1.0, d)
    parts = [num_ref[h] / dsafe[:, h][:, None] for h in range(H)]
    out_ref[...] = jnp.concatenate(parts, axis=-1)


def _tc_final(num, den):
    blk = 1000
    return pl.pallas_call(
        _tc_final_body,
        grid=(N // blk,),
        in_specs=[
            pl.BlockSpec((H, blk, DH), lambda i: (0, i, 0)),
            pl.BlockSpec((blk, H * TPH), lambda i: (i, 0)),
        ],
        out_specs=pl.BlockSpec((blk, H * DH), lambda i: (i, 0)),
        out_shape=jax.ShapeDtypeStruct((N, H * DH), jnp.float32),
    )(num, den)


def kernel(x, edge_indices, W, src_attn, dst_attn):
    row = edge_indices[0]
    col = edge_indices[1]
    wt = W.T
    sa = src_attn.reshape(H, DH)
    da = dst_attn.reshape(H, DH)
    h_heads, scores_nk = _tc_prep(x, wt, sa, da)
    scores = scores_nk.T.reshape(2, H, N)
    h_flat = h_heads.reshape(H * N, DH)
    heads = jnp.arange(H, dtype=jnp.int32)
    nblks = E // SUB // (SC_G * NSUB)
    colg = (col[None, :] + heads[:, None] * N).reshape(H, nblks, SC_G * NSUB, SUB)
    rows_s = (row[None, :] + (heads[:, None] % TPH) * N).reshape(H, nblks, SC_G * NSUB, SUB)
    num, den = _sc_edge(h_flat, scores, colg, rows_s)
    den_nk = den.reshape(H * TPH, N).T
    return _tc_final(num.reshape(H, N, DH), den_nk)


# speedup vs baseline: 1.8641x; 1.0054x over previous
import functools

import jax
import jax.numpy as jnp
from jax import lax
from jax.experimental import pallas as pl
from jax.experimental.pallas import tpu as pltpu
from jax.experimental.pallas import tpu_sc as plsc

N = 10000
E = 160000
D_IN = 256
H = 8
DH = 32

NC = 2
NS = 16
TPH = 4
HPC = H // NC
EPT = E // TPH
CH = 160
NCHUNK = EPT // CH
SUB = 80
NSUB = CH // SUB
GPS = SUB // 16
SC_G = 10
NSUPER = NCHUNK // SC_G
NB = N // 16
ZROWS = 50
OROWS = 500
ACC_ROWS_PER_TILE = HPC * N // NS


def _tc_prep_body(x_ref, wt_ref, sa_ref, da_ref, hh_ref, sc_ref):
    hb = jnp.dot(x_ref[...], wt_ref[...], preferred_element_type=jnp.float32)
    cols = []
    for h in range(H):
        hslice = hb[:, h * DH:(h + 1) * DH]
        hh_ref[h] = hslice
        cols.append((hslice * sa_ref[h][None, :]).sum(axis=-1))
    for h in range(H):
        hslice = hb[:, h * DH:(h + 1) * DH]
        cols.append((hslice * da_ref[h][None, :]).sum(axis=-1))
    sc_ref[...] = jnp.stack(cols, axis=-1)


def _tc_prep(x, wt, sa, da):
    blk = 1000
    return pl.pallas_call(
        _tc_prep_body,
        grid=(N // blk,),
        in_specs=[
            pl.BlockSpec((blk, D_IN), lambda i: (i, 0)),
            pl.BlockSpec((D_IN, D_IN), lambda i: (0, 0)),
            pl.BlockSpec((H, DH), lambda i: (0, 0)),
            pl.BlockSpec((H, DH), lambda i: (0, 0)),
        ],
        out_specs=[
            pl.BlockSpec((H, blk, DH), lambda i: (0, i, 0)),
            pl.BlockSpec((blk, 2 * H), lambda i: (i, 0)),
        ],
        out_shape=[
            jax.ShapeDtypeStruct((H, N, DH), jnp.float32),
            jax.ShapeDtypeStruct((N, 2 * H), jnp.float32),
        ],
    )(x, wt, sa, da)


def _sc_body(h_flat, scores, colg, rows_s, num_out, den_out,
             src_t, dst_t, den_t, idxg, idxs, e_v, rows, zbuf, acc,
             semg0, semg1, sems0, sems1):
    c = lax.axis_index("c")
    s = lax.axis_index("s")
    hl = s // TPH
    q = lax.rem(s, TPH)
    head = c * HPC + hl
    zero16 = jnp.zeros((16,), jnp.float32)
    headn = jnp.full((16,), head * N, jnp.int32)
    hln = jnp.full((16,), hl * N, jnp.int32)

    pltpu.sync_copy(scores.at[0, head], src_t)
    pltpu.sync_copy(scores.at[1, head], dst_t)

    gdn = lax.GatherDimensionNumbers(offset_dims=(), collapsed_slice_dims=(0,),
                                     start_index_map=(0,))
    def _perm(v, perm):
        return lax.gather(v, perm[:, None], gdn, slice_sizes=(1,),
                          mode=lax.GatherScatterMode.PROMISE_IN_BOUNDS)
    def _allmax(v):
        for k in (8, 4, 2, 1):
            perm = lax.iota(jnp.int32, 16) ^ k
            v = jnp.maximum(v, _perm(v, perm))
        return v
    def _mx(i, m):
        return jnp.maximum(m, src_t[pl.ds(i * 16, 16)])
    smax = _allmax(lax.fori_loop(0, NB, _mx, jnp.full((16,), -jnp.inf, jnp.float32)))
    def _mx2(i, m):
        return jnp.maximum(m, dst_t[pl.ds(i * 16, 16)])
    dmax = _allmax(lax.fori_loop(0, NB, _mx2, jnp.full((16,), -jnp.inf, jnp.float32)))
    shift = jnp.maximum(smax + dmax, 0.0)

    def _zd(i, _):
        den_t[pl.ds(i * 16, 16)] = zero16
        return 0
    lax.fori_loop(0, NB, _zd, 0)

    def _zb(i, _):
        zbuf[i, pl.ds(0, 16)] = zero16
        zbuf[i, pl.ds(16, 16)] = zero16
        return 0
    lax.fori_loop(0, ZROWS, _zb, 0)
    def _zacc(i, _):
        pltpu.async_copy(
            zbuf, acc.at[pl.ds(s * ACC_ROWS_PER_TILE + i * ZROWS, ZROWS)], semg0)
        return 0
    lax.fori_loop(0, ACC_ROWS_PER_TILE // ZROWS, _zacc, 0)
    def _zacc_w(i, _):
        pltpu.make_async_copy(
            zbuf, acc.at[pl.ds(s * ACC_ROWS_PER_TILE + i * ZROWS, ZROWS)],
            semg0).wait()
        return 0
    lax.fori_loop(0, ACC_ROWS_PER_TILE // ZROWS, _zacc_w, 0)
    plsc.subcore_barrier()

    semg = (semg0, semg1)
    sems = (sems0, sems1)

    def _load_idx(pb, sbl):
        pltpu.sync_copy(colg.at[head, sbl], idxg.at[pb])
        pltpu.sync_copy(rows_s.at[head, sbl], idxs.at[pb])

    def _fire_g(pd, pb, lci):
        for j in range(NSUB):
            pltpu.async_copy(h_flat.at[idxg.at[pb, lci * NSUB + j]],
                             rows.at[pd, j], semg[pd])

    def _drain_g(pd, pb, lci):
        for j in range(NSUB):
            pltpu.make_async_copy(h_flat.at[idxg.at[pb, lci * NSUB + j]],
                                  rows.at[pd, j], semg[pd]).wait()

    def _fire_s(pd, pb, lci):
        for j in range(NSUB):
            pltpu.async_copy(rows.at[pd, j],
                             acc.at[idxs.at[pb, lci * NSUB + j]], sems[pd],
                             add=True)

    def _drain_s(pd, pb, lci):
        for j in range(NSUB):
            pltpu.make_async_copy(rows.at[pd, j],
                                  acc.at[idxs.at[pb, lci * NSUB + j]],
                                  sems[pd]).wait()

    def _score(pb, lci):
        def _score_a(a, _a):
            def _score_t(t, _t):
                rs = idxs[pb, lci * NSUB + a, pl.ds(t * 16, 16)]
                cg = idxg[pb, lci * NSUB + a, pl.ds(t * 16, 16)]
                r = rs - hln
                col16 = cg - headn
                sv = plsc.load_gather(src_t, [r])
                dv = plsc.load_gather(dst_t, [col16])
                t_ = sv + dv
                sc = jnp.maximum(t_, 0.2 * t_) - shift
                e = jnp.exp(sc)
                e_v[pl.ds(a * SUB + t * 16, 16)] = e
                plsc.addupdate_scatter(den_t, [r], e)
                return 0
            lax.fori_loop(0, GPS, _score_t, 0)
            return 0
        lax.fori_loop(0, NSUB, _score_a, 0)

    def _scale(pd):
        for a in range(NSUB):
            def _scale_b(bb, _b, _a=a):
                eb = plsc.load_gather(
                    e_v, [jnp.full((16,), _a * SUB + bb, jnp.int32)])
                rows[pd, _a, bb, pl.ds(0, 16)] = rows[pd, _a, bb, pl.ds(0, 16)] * eb
                rows[pd, _a, bb, pl.ds(16, 16)] = rows[pd, _a, bb, pl.ds(16, 16)] * eb
                return 0
            lax.fori_loop(0, SUB, _scale_b, 0, unroll=4)

    def _step(pb, lci, sbl_next, first=False):
        pd = lci % 2
        po = 1 - pd
        _score(pb, lci)
        _drain_g(pd, pb, lci)
        _scale(pd)
        _fire_s(pd, pb, lci)
        if not first:
            prev_pb, prev_lci = (pb, lci - 1) if lci >= 1 else (1 - pb, SC_G - 1)
            _drain_s(po, prev_pb, prev_lci)
        if lci == SC_G - 2:
            _load_idx(1 - pb, sbl_next)
        next_pb, next_lci = (pb, lci + 1) if lci < SC_G - 1 else (1 - pb, 0)
        _fire_g(po, next_pb, next_lci)

    sbase = q * NSUPER
    _load_idx(0, sbase)
    _fire_g(0, 0, 0)
    for lci in range(SC_G):
        _step(0, lci, sbase + 1, first=(lci == 0))

    def _super(sb_off, _):
        sb = sb_off + 1
        pb = lax.rem(sb, 2)
        nxt = sbase + jnp.minimum(sb + 1, NSUPER - 1)
        for lci in range(SC_G):
            _step(pb, lci, nxt)
        return 0
    lax.fori_loop(0, NSUPER - 1, _super, 0)

    _drain_s(1, 0, SC_G - 1)
    _drain_g(0, 1, 0)

    plsc.subcore_barrier()

    out0 = hl * N + q * (N // TPH)
    nblk = N // TPH // OROWS
    def _out(i, _):
        pltpu.async_copy(acc.at[pl.ds(out0 + i * OROWS, OROWS)],
                         num_out.at[head, q * nblk + i], semg1)
        return 0
    lax.fori_loop(0, nblk, _out, 0)
    pltpu.sync_copy(den_t, den_out.at[head, q, 0])
    def _out_w(i, _):
        pltpu.make_async_copy(acc.at[pl.ds(out0 + i * OROWS, OROWS)],
                              num_out.at[head, q * nblk + i], semg1).wait()
        return 0
    lax.fori_loop(0, nblk, _out_w, 0)


def _sc_edge(h_flat, scores, colg, rows_s):
    mesh = plsc.VectorSubcoreMesh(core_axis_name="c", subcore_axis_name="s",
                                  num_cores=NC, num_subcores=NS)
    f = functools.partial(
        pl.kernel,
        compiler_params=pltpu.CompilerParams(use_tc_tiling_on_sc=False,
                                             needs_layout_passes=False),
        out_type=[
            jax.ShapeDtypeStruct((H, N // OROWS, OROWS, DH), jnp.float32),
            jax.ShapeDtypeStruct((H, TPH, 1, N), jnp.float32),
        ],
        mesh=mesh,
        scratch_types=[
            pltpu.VMEM((N,), jnp.float32),
            pltpu.VMEM((N,), jnp.float32),
            pltpu.VMEM((N,), jnp.float32),
            pltpu.VMEM((2, SC_G * NSUB, SUB), jnp.int32),
            pltpu.VMEM((2, SC_G * NSUB, SUB), jnp.int32),
            pltpu.VMEM((CH,), jnp.float32),
            pltpu.VMEM((2, NSUB, SUB, DH), jnp.float32),
            pltpu.VMEM((ZROWS, DH), jnp.float32),
            pltpu.VMEM_SHARED((HPC * N, DH), jnp.float32),
            pltpu.SemaphoreType.DMA,
            pltpu.SemaphoreType.DMA,
            pltpu.SemaphoreType.DMA,
            pltpu.SemaphoreType.DMA,
        ],
    )(_sc_body)
    return f(h_flat, scores, colg, rows_s)


def _tc_final_body(num_ref, den_ref, out_ref):
    blk = out_ref.shape[0]
    d = den_ref[...].reshape(blk, H, TPH).sum(axis=-1)
    dsafe = jnp.where(d == 0.0, 1.0, d)
    parts = [num_ref[h] / dsafe[:, h][:, None] for h in range(H)]
    out_ref[...] = jnp.concatenate(parts, axis=-1)


def _tc_final(num, den):
    blk = 1000
    return pl.pallas_call(
        _tc_final_body,
        grid=(N // blk,),
        in_specs=[
            pl.BlockSpec((H, blk, DH), lambda i: (0, i, 0)),
            pl.BlockSpec((blk, H * TPH), lambda i: (i, 0)),
        ],
        out_specs=pl.BlockSpec((blk, H * DH), lambda i: (i, 0)),
        out_shape=jax.ShapeDtypeStruct((N, H * DH), jnp.float32),
    )(num, den)


def kernel(x, edge_indices, W, src_attn, dst_attn):
    row = edge_indices[0]
    col = edge_indices[1]
    wt = W.T
    sa = src_attn.reshape(H, DH)
    da = dst_attn.reshape(H, DH)
    h_heads, scores_nk = _tc_prep(x, wt, sa, da)
    scores = scores_nk.T.reshape(2, H, N)
    h_flat = h_heads.reshape(H * N, DH)
    heads = jnp.arange(H, dtype=jnp.int32)
    nblks = E // SUB // (SC_G * NSUB)
    colg = (col[None, :] + heads[:, None] * N).reshape(H, nblks, SC_G * NSUB, SUB)
    rows_s = (row[None, :] + (heads[:, None] % TPH) * N).reshape(H, nblks, SC_G * NSUB, SUB)
    num, den = _sc_edge(h_flat, scores, colg, rows_s)
    den_nk = den.reshape(H * TPH, N).T
    return _tc_final(num.reshape(H, N, DH), den_nk)
